# Initial kernel scaffold; baseline (speedup 1.0000x reference)
import jax
import jax.numpy as jnp
from jax.experimental import pallas as pl


def kernel(x, indices, w0, scale0, bias0, w1, scale1, bias1):
    raise NotImplementedError("write your pallas kernel here")


# speedup vs baseline: 1.3288x; 1.3288x over previous
import functools

import jax
import jax.numpy as jnp
from jax import lax
from jax.experimental import pallas as pl
from jax.experimental.pallas import tpu as pltpu


def _decoder_body(x_ref, i_ref, w0_ref, s0_ref, b0_ref, w1_ref, s1_ref,
                  b1_ref, o_ref, *, H, W):
    HW = H * W
    xv = x_ref[0]
    iv = i_ref[0]
    lane = lax.broadcasted_iota(jnp.int32, xv.shape, 1)
    ih = lane // W
    iw = lane - W * ih
    base = (2 * ih) * (2 * W) + 2 * iw

    phases = {}
    for py in range(2):
        for px in range(2):
            m = jnp.where(iv == base + (py * 2 * W + px), xv, 0.0)
            phases[(py, px)] = jnp.transpose(m.astype(jnp.bfloat16))

    col_id = lax.broadcasted_iota(jnp.int32, (HW, 1), 0) % W
    not_left = col_id != 0
    not_right = col_id != (W - 1)

    def conv_layer(ph, wm, sc, bi, last):
        cin = wm.shape[0] // 9
        zpad = jnp.zeros((W + 1, cin), jnp.bfloat16)
        ap = {k: jnp.concatenate([zpad, v, zpad], axis=0)
              for k, v in ph.items()}
        out = {}
        for py in range(2):
            for px in range(2):
                taps = []
                for oy in (-1, 0, 1):
                    sy = (py + oy) % 2
                    gy = (py + oy) // 2
                    for ox in (-1, 0, 1):
                        sx = (px + ox) % 2
                        gx = (px + ox) // 2
                        s = (W + 1) + gy * W + gx
                        t = ap[(sy, sx)][s:s + HW, :]
                        if gx == -1:
                            t = jnp.where(not_left, t, 0)
                        elif gx == 1:
                            t = jnp.where(not_right, t, 0)
                        taps.append(t)
                col = jnp.concatenate(taps, axis=1)
                h = jnp.dot(col, wm, preferred_element_type=jnp.float32)
                h = jnp.maximum(h * sc + bi, 0.0)
                out[(py, px)] = h if last else h.astype(jnp.bfloat16)
        return out

    p1 = conv_layer(phases, w0_ref[...], s0_ref[...], b0_ref[...], False)
    p2 = conv_layer(p1, w1_ref[...], s1_ref[...], b1_ref[...], True)
    for py in range(2):
        for px in range(2):
            o_ref[0, 2 * py + px] = p2[(py, px)]


def kernel(x, indices, w0, scale0, bias0, w1, scale1, bias1):
    N, Cin, H, W = x.shape
    HW = H * W
    C1 = w0.shape[3]
    C2 = w1.shape[3]
    xr = x.astype(jnp.float32).reshape(N, Cin, HW)
    ir = indices.astype(jnp.int32).reshape(N, Cin, HW)
    wm0 = w0.reshape(9 * Cin, C1).astype(jnp.bfloat16)
    wm1 = w1.reshape(9 * C1, C2).astype(jnp.bfloat16)
    s0 = scale0.reshape(1, C1)
    b0 = bias0.reshape(1, C1)
    s1 = scale1.reshape(1, C2)
    b1 = bias1.reshape(1, C2)

    out = pl.pallas_call(
        functools.partial(_decoder_body, H=H, W=W),
        out_shape=jax.ShapeDtypeStruct((N, 4, HW, C2), jnp.float32),
        grid=(N,),
        in_specs=[
            pl.BlockSpec((1, Cin, HW), lambda n: (n, 0, 0)),
            pl.BlockSpec((1, Cin, HW), lambda n: (n, 0, 0)),
            pl.BlockSpec((9 * Cin, C1), lambda n: (0, 0)),
            pl.BlockSpec((1, C1), lambda n: (0, 0)),
            pl.BlockSpec((1, C1), lambda n: (0, 0)),
            pl.BlockSpec((9 * C1, C2), lambda n: (0, 0)),
            pl.BlockSpec((1, C2), lambda n: (0, 0)),
            pl.BlockSpec((1, C2), lambda n: (0, 0)),
        ],
        out_specs=pl.BlockSpec((1, 4, HW, C2), lambda n: (n, 0, 0, 0)),
        compiler_params=pltpu.CompilerParams(
            dimension_semantics=("parallel",)),
    )(xr, ir, wm0, s0, b0, wm1, s1, b1)

    y = out.reshape(N, 2, 2, H, W, C2).transpose(0, 5, 3, 1, 4, 2)
    return y.reshape(N, C2, 2 * H, 2 * W)
